# baseline (speedup 1.0000x reference)
import functools
import math

import jax
import jax.numpy as jnp
import numpy as np
from jax import lax
from jax.experimental import pallas as pl
from jax.experimental.pallas import tpu as pltpu
from jax.experimental.pallas import tpu_sc as plsc

VOCAB = 1_000_000
D = 64
B = 4096
S = 200
NC = 2
NS = 16
NW = NC * NS
ROWS_PER_W = B * S // NW
CHUNKS = ROWS_PER_W // S
HALF = S // 2
NBUF = 4
LOOKAHEAD = 2
LANES = 16


def _positional_table() -> np.ndarray:
    dims = np.arange(0, D, 2, dtype=np.float32)
    freq = np.exp(dims * (-math.log(10000.0) / D))
    pos = np.arange(S, dtype=np.float32)[:, None]
    pe = np.zeros((S, D), dtype=np.float32)
    pe[:, 0::2] = np.sin(pos * freq)
    pe[:, 1::2] = np.cos(pos * freq)
    return pe


_PE = _positional_table()

_mesh = plsc.VectorSubcoreMesh(core_axis_name="c", subcore_axis_name="s")


@functools.partial(
    pl.kernel,
    out_type=jax.ShapeDtypeStruct((B * S, D), jnp.float32),
    mesh=_mesh,
    compiler_params=pltpu.CompilerParams(use_tc_tiling_on_sc=False),
    scratch_types=[
        pltpu.VMEM((2 * CHUNKS, HALF), jnp.int32),
        pltpu.VMEM((S, D), jnp.float32),
        pltpu.VMEM((S, D), jnp.float32),
        pltpu.VMEM((S, D), jnp.float32),
        pltpu.VMEM((S, D), jnp.float32),
        pltpu.VMEM((S, D), jnp.float32),
        pltpu.SemaphoreType.DMA,
        pltpu.SemaphoreType.DMA,
        pltpu.SemaphoreType.DMA,
        pltpu.SemaphoreType.DMA,
        pltpu.SemaphoreType.DMA,
        pltpu.SemaphoreType.DMA,
        pltpu.SemaphoreType.DMA,
        pltpu.SemaphoreType.DMA,
    ],
)
def _embed_kernel(x_hbm, table_hbm, pe_hbm, out_hbm, idx_v, pe_v,
                  b0, b1, b2, b3, g0, g1, g2, g3, s0, s1, s2, s3):
    bufs = (b0, b1, b2, b3)
    gsems = (g0, g1, g2, g3)
    ssems = (s0, s1, s2, s3)

    wid = lax.axis_index("s") * NC + lax.axis_index("c")
    base_row = wid * ROWS_PER_W

    pltpu.sync_copy(x_hbm.at[wid], idx_v)
    pltpu.sync_copy(pe_hbm, pe_v)

    def start_gather(c, slot):
        pltpu.async_copy(table_hbm.at[idx_v.at[2 * c]],
                         bufs[slot].at[pl.ds(0, HALF)], gsems[slot])
        pltpu.async_copy(table_hbm.at[idx_v.at[2 * c + 1]],
                         bufs[slot].at[pl.ds(HALF, HALF)], gsems[slot])

    def wait_gather(slot):
        pltpu.make_async_copy(table_hbm.at[pl.ds(0, S)], bufs[slot],
                              gsems[slot]).wait()

    def start_scatter(c, slot):
        pltpu.async_copy(bufs[slot],
                         out_hbm.at[pl.ds(base_row + c * S, S)], ssems[slot])

    def wait_scatter(slot):
        pltpu.make_async_copy(bufs[slot], out_hbm.at[pl.ds(0, S)],
                              ssems[slot]).wait()

    def add_pe(slot):
        buf = bufs[slot]

        @plsc.parallel_loop(0, S, unroll=4)
        def _row(r):
            for j in range(D // LANES):
                sl = pl.ds(j * LANES, LANES)
                plsc.addupdate(buf.at[r, sl], pe_v[r, sl])

    for p in range(LOOKAHEAD):
        start_gather(p, p)

    @pl.loop(0, CHUNKS, step=NBUF)
    def _group(g):
        for b in range(NBUF):
            c = g + b
            slot_la = (b + LOOKAHEAD) % NBUF
            c_la = c + LOOKAHEAD

            @pl.when(c_la >= NBUF)
            def _():
                wait_scatter(slot_la)

            @pl.when(c_la < CHUNKS)
            def _():
                start_gather(c_la, slot_la)

            wait_gather(b)
            add_pe(b)
            start_scatter(c, b)

    for c in range(CHUNKS - LOOKAHEAD, CHUNKS):
        wait_scatter(c % NBUF)


def kernel(x, table):
    x3 = x.astype(jnp.int32).reshape(NW, 2 * CHUNKS, HALF)
    out = _embed_kernel(x3, table, jnp.asarray(_PE))
    return out.reshape(B, S, D)
